# baseline (speedup 1.0000x reference)
import functools

import jax
import jax.numpy as jnp
from jax import lax
from jax.experimental import pallas as pl
from jax.experimental.pallas import tpu as pltpu
from jax.experimental.pallas import tpu_sc as plsc

_B, _D, _H, _W = 8, 256, 32, 32
_NW = 32
_CPW = (2 * _D) // _NW
_L = 16


def _pos_kernel(row_hbm, col_hbm, out_hbm, tab_v, chunk_v, sem):
    cid = lax.axis_index("c")
    sid = lax.axis_index("s")
    wid = sid * 2 + cid

    pltpu.sync_copy(col_hbm.at[pl.ds(0, _W)], tab_v.at[pl.ds(0, _W)])
    pltpu.sync_copy(row_hbm.at[pl.ds(0, _H)], tab_v.at[pl.ds(_W, _H)])

    c0 = wid * _CPW
    lane = lax.iota(jnp.int32, _L)

    @pl.when(wid < _NW // 2)
    def _col_half():
        for cl in range(_CPW):
            ch = jnp.full((_L,), cl, jnp.int32) + c0
            v0 = plsc.load_gather(tab_v, [lane, ch])
            v1 = plsc.load_gather(tab_v, [lane + _L, ch])

            def body(i, carry, cl=cl, v0=v0, v1=v1):
                chunk_v[cl, i, pl.ds(0, _L)] = v0
                chunk_v[cl, i, pl.ds(_L, _L)] = v1
                return carry

            lax.fori_loop(0, _H, body, 0)

    @pl.when(wid >= _NW // 2)
    def _row_half():
        for cl in range(_CPW):
            ch = jnp.full((_L,), cl - _D, jnp.int32) + c0

            def body(i, carry, ch=ch):
                idx0 = jnp.full((_L,), _H, jnp.int32) + i
                v = plsc.load_gather(tab_v, [idx0, ch])
                chunk_v[cl, i, pl.ds(0, _L)] = v
                chunk_v[cl, i, pl.ds(_L, _L)] = v
                return carry

            lax.fori_loop(0, _H, body, 0)

    copies = [
        pltpu.async_copy(chunk_v, out_hbm.at[b, pl.ds(c0, _CPW)], sem)
        for b in range(_B)
    ]
    for c in copies:
        c.wait()


_mesh = plsc.VectorSubcoreMesh(core_axis_name="c", subcore_axis_name="s")

_pos_call = functools.partial(
    pl.kernel,
    mesh=_mesh,
    out_type=jax.ShapeDtypeStruct((_B, 2 * _D, _H, _W), jnp.float32),
    scratch_types=[
        pltpu.VMEM((2 * _W, _D), jnp.float32),
        pltpu.VMEM((_CPW, _H, _W), jnp.float32),
        pltpu.SemaphoreType.DMA,
    ],
    compiler_params=pltpu.CompilerParams(
        use_tc_tiling_on_sc=False, needs_layout_passes=False
    ),
)(_pos_kernel)


def kernel(x, row_embed, col_embed):
    del x
    return _pos_call(row_embed, col_embed)


# speedup vs baseline: 3.5445x; 3.5445x over previous
import functools

import jax
import jax.numpy as jnp
from jax import lax
from jax.experimental import pallas as pl
from jax.experimental.pallas import tpu as pltpu
from jax.experimental.pallas import tpu_sc as plsc

_B, _D, _H, _W = 8, 256, 32, 32
_L = 16


def _pos_kernel(row_hbm, col_hbm, out_hbm, colv, rowv, block, sem):
    cid = lax.axis_index("c")
    sid = lax.axis_index("s")
    i = sid * 2 + cid

    pltpu.sync_copy(col_hbm.at[pl.ds(0, _W)], colv)
    pltpu.sync_copy(row_hbm.at[pl.ds(i, 1)], rowv)

    for jt in range(4):
        for ct in range(2):
            for jr in range(8):
                for v in range(8):
                    block[jt, ct, jr, pl.ds(v * _L, _L)] = (
                        colv[jt * 8 + jr, pl.ds(ct * 128 + v * _L, _L)]
                    )
    for ct in range(2):
        for v in range(8):
            rv = rowv[0, pl.ds(ct * 128 + v * _L, _L)]
            for jt in range(4):
                for jr in range(8):
                    block[jt, 2 + ct, jr, pl.ds(v * _L, _L)] = rv

    copies = [
        pltpu.async_copy(block, out_hbm.at[b, i], sem) for b in range(_B)
    ]
    for c in copies:
        c.wait()


_mesh = plsc.VectorSubcoreMesh(core_axis_name="c", subcore_axis_name="s")

_pos_call = functools.partial(
    pl.kernel,
    mesh=_mesh,
    out_type=jax.ShapeDtypeStruct((_B, _H, 4, 4, 8, 128), jnp.float32),
    scratch_types=[
        pltpu.VMEM((_W, _D), jnp.float32),
        pltpu.VMEM((1, _D), jnp.float32),
        pltpu.VMEM((4, 4, 8, 128), jnp.float32),
        pltpu.SemaphoreType.DMA,
    ],
    compiler_params=pltpu.CompilerParams(
        use_tc_tiling_on_sc=False, needs_layout_passes=False
    ),
)(_pos_kernel)


def kernel(x, row_embed, col_embed):
    del x
    out6 = _pos_call(row_embed, col_embed)
    return out6.transpose((0, 3, 5, 1, 2, 4)).reshape(_B, 2 * _D, _H, _W)


# speedup vs baseline: 3.5457x; 1.0004x over previous
import functools

import jax
import jax.numpy as jnp
from jax import lax
from jax.experimental import pallas as pl
from jax.experimental.pallas import tpu as pltpu
from jax.experimental.pallas import tpu_sc as plsc

_B, _D, _H, _W = 8, 256, 32, 32
_L = 16


def _pos_kernel(row_hbm, col_hbm, out_hbm, colv, rowv8, block, sem):
    cid = lax.axis_index("c")
    sid = lax.axis_index("s")
    i = sid * 2 + cid

    pltpu.sync_copy(col_hbm.at[pl.ds(0, _W)], colv)
    pltpu.sync_copy(row_hbm.at[pl.ds((i // 8) * 8, 8)], rowv8)
    ir = i % 8

    def col_body(t, carry):
        jt = t // 16
        ct = (t // 8) % 2
        jr = t % 8
        for v in range(8):
            block[jt, ct, jr, pl.ds(v * _L, _L)] = (
                colv[jt * 8 + jr, pl.ds(ct * 128 + v * _L, _L)]
            )
        return carry

    lax.fori_loop(0, 64, col_body, 0)

    rvs = [rowv8[ir, pl.ds(k * _L, _L)] for k in range(16)]

    def row_body(t, carry):
        jt = t // 8
        jr = t % 8
        for k in range(16):
            block[jt, 2 + k // 8, jr, pl.ds((k % 8) * _L, _L)] = rvs[k]
        return carry

    lax.fori_loop(0, 32, row_body, 0)

    copies = [
        pltpu.async_copy(block, out_hbm.at[b, i], sem) for b in range(_B)
    ]
    for c in copies:
        c.wait()


_mesh = plsc.VectorSubcoreMesh(core_axis_name="c", subcore_axis_name="s")

_pos_call = functools.partial(
    pl.kernel,
    mesh=_mesh,
    out_type=jax.ShapeDtypeStruct((_B, _H, 4, 4, 8, 128), jnp.float32),
    scratch_types=[
        pltpu.VMEM((_W, _D), jnp.float32),
        pltpu.VMEM((8, _D), jnp.float32),
        pltpu.VMEM((4, 4, 8, 128), jnp.float32),
        pltpu.SemaphoreType.DMA,
    ],
    compiler_params=pltpu.CompilerParams(
        use_tc_tiling_on_sc=True, needs_layout_passes=False
    ),
)(_pos_kernel)


def kernel(x, row_embed, col_embed):
    del x
    out6 = _pos_call(row_embed, col_embed)
    return out6.transpose((0, 3, 5, 1, 2, 4)).reshape(_B, 2 * _D, _H, _W)


# speedup vs baseline: 7.2738x; 2.0514x over previous
import functools

import jax
import jax.numpy as jnp
from jax.experimental import pallas as pl

_B, _D, _H, _W = 8, 256, 32, 32
_RG = 8


def _pos_body(colv, rowv, out_ref):
    ig = pl.program_id(1)
    cols = colv[...]
    rows = rowv[pl.ds(ig * _RG, _RG), :]
    out_ref[0, :, :, 0:_D] = jnp.broadcast_to(cols[None], (_RG, _W, _D))
    out_ref[0, :, :, _D : 2 * _D] = jnp.broadcast_to(
        rows[:, None, :], (_RG, _W, _D)
    )


_pos_call = functools.partial(
    pl.pallas_call,
    grid=(_B, _H // _RG),
    in_specs=[
        pl.BlockSpec((_W, _D), lambda b, ig: (0, 0)),
        pl.BlockSpec((_H, _D), lambda b, ig: (0, 0)),
    ],
    out_specs=pl.BlockSpec(
        (1, _RG, _W, 2 * _D), lambda b, ig: (b, ig, 0, 0)
    ),
    out_shape=jax.ShapeDtypeStruct((_B, _H, _W, 2 * _D), jnp.float32),
)(_pos_body)


def kernel(x, row_embed, col_embed):
    del x
    out = _pos_call(col_embed, row_embed)
    return out.transpose((0, 3, 1, 2))


# speedup vs baseline: 15.3523x; 2.1106x over previous
import functools

import jax
import jax.numpy as jnp
from jax.experimental import pallas as pl
from jax.experimental.pallas import tpu as pltpu

_B, _D, _H, _W = 8, 256, 32, 32


def _pos_body(colv, rowv, out_ref, slab, sem):
    cols = colv[...]
    rows = rowv[...]
    slab[:, :, 0:_D] = jnp.broadcast_to(cols[None], (_H, _W, _D))
    slab[:, :, _D : 2 * _D] = jnp.broadcast_to(rows[:, None, :], (_H, _W, _D))
    copies = [
        pltpu.make_async_copy(slab, out_ref.at[b], sem) for b in range(_B)
    ]
    for c in copies:
        c.start()
    for c in copies:
        c.wait()


_pos_call = functools.partial(
    pl.pallas_call,
    grid=(1,),
    in_specs=[
        pl.BlockSpec((_W, _D), lambda g: (0, 0)),
        pl.BlockSpec((_H, _D), lambda g: (0, 0)),
    ],
    out_specs=pl.BlockSpec(memory_space=pltpu.MemorySpace.HBM),
    out_shape=jax.ShapeDtypeStruct((_B, _H, _W, 2 * _D), jnp.float32),
    scratch_shapes=[
        pltpu.VMEM((_H, _W, 2 * _D), jnp.float32),
        pltpu.SemaphoreType.DMA,
    ],
)(_pos_body)


def kernel(x, row_embed, col_embed):
    del x
    out = _pos_call(col_embed, row_embed)
    return out.transpose((0, 3, 1, 2))


# speedup vs baseline: 15.4203x; 1.0044x over previous
import functools

import jax
import jax.numpy as jnp
from jax.experimental import pallas as pl
from jax.experimental.pallas import tpu as pltpu

_B, _D, _H, _W = 8, 256, 32, 32


def _pos_body(colv, rowv, out_ref, slab, sem):
    cols = colv[...]
    rows = rowv[...]
    slab[:, :, 0:_D] = jnp.broadcast_to(cols[None], (_H, _W, _D))
    slab[:, :, _D : 2 * _D] = jnp.broadcast_to(rows[:, None, :], (_H, _W, _D))
    copies = [
        pltpu.make_async_copy(
            slab.at[pl.ds(h * (_H // 2), _H // 2)],
            out_ref.at[b, pl.ds(h * (_H // 2), _H // 2)],
            sem,
        )
        for b in range(_B)
        for h in range(2)
    ]
    for c in copies:
        c.start()
    for c in copies:
        c.wait()


_pos_call = functools.partial(
    pl.pallas_call,
    grid=(1,),
    in_specs=[
        pl.BlockSpec((_W, _D), lambda g: (0, 0)),
        pl.BlockSpec((_H, _D), lambda g: (0, 0)),
    ],
    out_specs=pl.BlockSpec(memory_space=pltpu.MemorySpace.HBM),
    out_shape=jax.ShapeDtypeStruct((_B, _H, _W, 2 * _D), jnp.float32),
    scratch_shapes=[
        pltpu.VMEM((_H, _W, 2 * _D), jnp.float32),
        pltpu.SemaphoreType.DMA,
    ],
)(_pos_body)


def kernel(x, row_embed, col_embed):
    del x
    out = _pos_call(col_embed, row_embed)
    return out.transpose((0, 3, 1, 2))


# speedup vs baseline: 15.7336x; 1.0203x over previous
import functools

import jax
import jax.numpy as jnp
from jax.experimental import pallas as pl
from jax.experimental.pallas import tpu as pltpu

_B, _D, _H, _W = 8, 256, 32, 32


def _pos_body(colv, rowv, out_ref, slab, sem):
    hh = _H // 2
    cols = colv[...]
    rows = rowv[...]
    copies = []
    for h in range(2):
        sl = pl.ds(h * hh, hh)
        slab[sl, :, 0:_D] = jnp.broadcast_to(cols[None], (hh, _W, _D))
        slab[sl, :, _D : 2 * _D] = jnp.broadcast_to(
            rows[h * hh : (h + 1) * hh][:, None, :], (hh, _W, _D)
        )
        for b in range(_B):
            c = pltpu.make_async_copy(slab.at[sl], out_ref.at[b, sl], sem)
            c.start()
            copies.append(c)
    for c in copies:
        c.wait()


_pos_call = functools.partial(
    pl.pallas_call,
    grid=(1,),
    in_specs=[
        pl.BlockSpec((_W, _D), lambda g: (0, 0)),
        pl.BlockSpec((_H, _D), lambda g: (0, 0)),
    ],
    out_specs=pl.BlockSpec(memory_space=pltpu.MemorySpace.HBM),
    out_shape=jax.ShapeDtypeStruct((_B, _H, _W, 2 * _D), jnp.float32),
    scratch_shapes=[
        pltpu.VMEM((_H, _W, 2 * _D), jnp.float32),
        pltpu.SemaphoreType.DMA,
    ],
)(_pos_body)


def kernel(x, row_embed, col_embed):
    del x
    out = _pos_call(col_embed, row_embed)
    return out.transpose((0, 3, 1, 2))


# speedup vs baseline: 15.8954x; 1.0103x over previous
import functools

import jax
import jax.numpy as jnp
from jax.experimental import pallas as pl
from jax.experimental.pallas import tpu as pltpu

_B, _D, _H, _W = 8, 256, 32, 32


def _pos_body(colv, rowv, out_ref, slab, sem):
    hh = _H // 4
    cols = colv[...]
    rows = rowv[...]
    copies = []
    for h in range(4):
        sl = pl.ds(h * hh, hh)
        slab[sl, :, 0:_D] = jnp.broadcast_to(cols[None], (hh, _W, _D))
        slab[sl, :, _D : 2 * _D] = jnp.broadcast_to(
            rows[h * hh : (h + 1) * hh][:, None, :], (hh, _W, _D)
        )
        for b in range(_B):
            c = pltpu.make_async_copy(slab.at[sl], out_ref.at[b, sl], sem)
            c.start()
            copies.append(c)
    for c in copies:
        c.wait()


_pos_call = functools.partial(
    pl.pallas_call,
    grid=(1,),
    in_specs=[
        pl.BlockSpec((_W, _D), lambda g: (0, 0)),
        pl.BlockSpec((_H, _D), lambda g: (0, 0)),
    ],
    out_specs=pl.BlockSpec(memory_space=pltpu.MemorySpace.HBM),
    out_shape=jax.ShapeDtypeStruct((_B, _H, _W, 2 * _D), jnp.float32),
    scratch_shapes=[
        pltpu.VMEM((_H, _W, 2 * _D), jnp.float32),
        pltpu.SemaphoreType.DMA,
    ],
)(_pos_body)


def kernel(x, row_embed, col_embed):
    del x
    out = _pos_call(col_embed, row_embed)
    return out.transpose((0, 3, 1, 2))


# speedup vs baseline: 16.0912x; 1.0123x over previous
import functools

import jax
import jax.numpy as jnp
from jax.experimental import pallas as pl
from jax.experimental.pallas import tpu as pltpu

_B, _D, _H, _W = 8, 256, 32, 32


def _pos_body(colv, rowv, out_ref, slab, sem):
    hh = _H // 8
    cols = colv[...]
    rows = rowv[...]
    copies = []
    for h in range(8):
        sl = pl.ds(h * hh, hh)
        slab[sl, :, 0:_D] = jnp.broadcast_to(cols[None], (hh, _W, _D))
        slab[sl, :, _D : 2 * _D] = jnp.broadcast_to(
            rows[h * hh : (h + 1) * hh][:, None, :], (hh, _W, _D)
        )
        for b in range(_B):
            c = pltpu.make_async_copy(slab.at[sl], out_ref.at[b, sl], sem)
            c.start()
            copies.append(c)
    for c in copies:
        c.wait()


_pos_call = functools.partial(
    pl.pallas_call,
    grid=(1,),
    in_specs=[
        pl.BlockSpec((_W, _D), lambda g: (0, 0)),
        pl.BlockSpec((_H, _D), lambda g: (0, 0)),
    ],
    out_specs=pl.BlockSpec(memory_space=pltpu.MemorySpace.HBM),
    out_shape=jax.ShapeDtypeStruct((_B, _H, _W, 2 * _D), jnp.float32),
    scratch_shapes=[
        pltpu.VMEM((_H, _W, 2 * _D), jnp.float32),
        pltpu.SemaphoreType.DMA,
    ],
)(_pos_body)


def kernel(x, row_embed, col_embed):
    del x
    out = _pos_call(col_embed, row_embed)
    return out.transpose((0, 3, 1, 2))
